# Initial kernel scaffold; baseline (speedup 1.0000x reference)
import jax
import jax.numpy as jnp
from jax.experimental import pallas as pl


def kernel(x, table_1, table_2):
    raise NotImplementedError("write your pallas kernel here")


# speedup vs baseline: 1.4688x; 1.4688x over previous
import functools

import jax
import jax.numpy as jnp
from jax import lax
from jax.experimental import pallas as pl
from jax.experimental.pallas import tpu as pltpu
from jax.experimental.pallas import tpu_sc as plsc

D = 32
NW = 32
CHUNK = 128
LANES = 16


def _make_sc_kernel(flat, nchunk):
    mesh = plsc.VectorSubcoreMesh(core_axis_name="c", subcore_axis_name="s")

    @functools.partial(
        pl.kernel,
        mesh=mesh,
        compiler_params=pltpu.CompilerParams(use_tc_tiling_on_sc=False),
        out_type=jax.ShapeDtypeStruct((NW, nchunk, CHUNK, D), jnp.float32),
        scratch_types=[
            pltpu.VMEM((nchunk, CHUNK), jnp.int32),
            pltpu.VMEM((CHUNK, D), jnp.float32),
            pltpu.VMEM((CHUNK, D), jnp.float32),
            pltpu.SemaphoreType.DMA,
            pltpu.SemaphoreType.DMA,
        ],
    )
    def sc_kernel(x_hbm, t1_hbm, t2_hbm, out_hbm, idx_v, a_v, b_v, sem1, sem2):
        wid = lax.axis_index("s") * 2 + lax.axis_index("c")
        pltpu.sync_copy(x_hbm.at[wid], idx_v)

        def chunk_body(j, carry):
            h1 = pltpu.async_copy(t1_hbm.at[idx_v.at[j]], a_v, sem1)
            h2 = pltpu.async_copy(t2_hbm.at[idx_v.at[j]], b_v, sem2)
            h1.wait()
            h2.wait()

            def row_body(i, c2):
                for c in range(D // LANES):
                    sl = pl.ds(c * LANES, LANES)
                    a_v[i, sl] = a_v[i, sl] + b_v[i, sl]
                return c2

            lax.fori_loop(0, CHUNK, row_body, 0)
            pltpu.sync_copy(a_v, out_hbm.at[wid, j])
            return carry

        lax.fori_loop(0, nchunk, chunk_body, 0)

    return sc_kernel


def kernel(x, table_1, table_2):
    rows, seq = x.shape
    flat = rows * seq
    per_w = flat // NW
    nchunk = per_w // CHUNK
    xf = x.reshape(NW, nchunk, CHUNK)
    out = _make_sc_kernel(flat, nchunk)(xf, table_1, table_2)
    return out.reshape(rows, seq, D)


# speedup vs baseline: 1.6252x; 1.1065x over previous
import functools

import jax
import jax.numpy as jnp
from jax import lax
from jax.experimental import pallas as pl
from jax.experimental.pallas import tpu as pltpu
from jax.experimental.pallas import tpu_sc as plsc

D = 32
NW = 32
CHUNK = 128
LANES = 16
NBUF = 4


def _make_sc_kernel(nchunk):
    mesh = plsc.VectorSubcoreMesh(core_axis_name="c", subcore_axis_name="s")
    nrounds = nchunk // NBUF

    @functools.partial(
        pl.kernel,
        mesh=mesh,
        compiler_params=pltpu.CompilerParams(use_tc_tiling_on_sc=False),
        out_type=jax.ShapeDtypeStruct((NW, nchunk, CHUNK, D), jnp.float32),
        scratch_types=[
            pltpu.VMEM((nchunk, CHUNK), jnp.int32),
            pltpu.VMEM((NBUF, CHUNK, D), jnp.float32),
            pltpu.VMEM((NBUF, CHUNK, D), jnp.float32),
            pltpu.VMEM((NBUF, CHUNK, D), jnp.float32),
        ] + [pltpu.SemaphoreType.DMA] * (3 * NBUF),
    )
    def sc_kernel(x_hbm, t1_hbm, t2_hbm, out_hbm, idx_v, a_v, b_v, c_v, *sems):
        sga = sems[0:NBUF]
        sgb = sems[NBUF:2 * NBUF]
        so = sems[2 * NBUF:3 * NBUF]

        wid = lax.axis_index("s") * 2 + lax.axis_index("c")
        pltpu.sync_copy(x_hbm.at[wid], idx_v)

        def issue_gathers(j, s):
            pltpu.async_copy(t1_hbm.at[idx_v.at[j]], a_v.at[s], sga[s])
            pltpu.async_copy(t2_hbm.at[idx_v.at[j]], b_v.at[s], sgb[s])

        for s in range(NBUF):
            issue_gathers(s, s)

        def round_body(g, carry):
            for s in range(NBUF):
                j = g * NBUF + s
                pltpu.make_async_copy(
                    t1_hbm.at[idx_v.at[j]], a_v.at[s], sga[s]).wait()
                pltpu.make_async_copy(
                    t2_hbm.at[idx_v.at[j]], b_v.at[s], sgb[s]).wait()

                def add_body(i, c2):
                    for col in range(D // LANES):
                        sl = pl.ds(col * LANES, LANES)
                        c_v[s, i, sl] = a_v[s, i, sl] + b_v[s, i, sl]
                    return c2

                lax.fori_loop(0, CHUNK, add_body, 0)

                @pl.when(g < nrounds - 1)
                def _():
                    issue_gathers(j + NBUF, s)

                @pl.when(g > 0)
                def _():
                    pltpu.make_async_copy(
                        c_v.at[s], out_hbm.at[wid, s], so[s]).wait()

                pltpu.async_copy(c_v.at[s], out_hbm.at[wid, j], so[s])
            return carry

        lax.fori_loop(0, nrounds, round_body, 0)
        for s in range(NBUF):
            pltpu.make_async_copy(c_v.at[s], out_hbm.at[wid, s], so[s]).wait()

    return sc_kernel


def kernel(x, table_1, table_2):
    rows, seq = x.shape
    flat = rows * seq
    per_w = flat // NW
    nchunk = per_w // CHUNK
    xf = x.reshape(NW, nchunk, CHUNK)
    out = _make_sc_kernel(nchunk)(xf, table_1, table_2)
    return out.reshape(rows, seq, D)


# speedup vs baseline: 3.3526x; 2.0628x over previous
import functools

import jax
import jax.numpy as jnp
from jax import lax
from jax.experimental import pallas as pl
from jax.experimental.pallas import tpu as pltpu
from jax.experimental.pallas import tpu_sc as plsc

D = 32
NW = 32
CHUNK = 128
LANES = 16
NBUF = 4
SEQ = 26
BATCH = 16384
IB = BATCH // CHUNK
TC_BK = 2048


def _tc_sum_transpose(t1t, t2t):
    v = t1t.shape[1]
    sub = TC_BK // 4
    nsteps = (v + TC_BK - 1) // TC_BK

    def body(a_ref, b_ref, o_ref):
        s = a_ref[...] + b_ref[...]
        o_ref[...] = jnp.concatenate(
            [s[:, u * sub:(u + 1) * sub].T for u in range(4)], axis=1)

    return pl.pallas_call(
        body,
        grid=(nsteps,),
        in_specs=[
            pl.BlockSpec((D, TC_BK), lambda i: (0, i)),
            pl.BlockSpec((D, TC_BK), lambda i: (0, i)),
        ],
        out_specs=pl.BlockSpec((sub, 4 * D), lambda i: (i, 0)),
        out_shape=jax.ShapeDtypeStruct((nsteps * sub, 4 * D), jnp.float32),
        compiler_params=pltpu.CompilerParams(
            dimension_semantics=("arbitrary",)),
    )(t1t, t2t)


def _make_sc_kernel(nchunk):
    mesh = plsc.VectorSubcoreMesh(core_axis_name="c", subcore_axis_name="s")
    nrounds = nchunk // NBUF

    @functools.partial(
        pl.kernel,
        mesh=mesh,
        compiler_params=pltpu.CompilerParams(
            use_tc_tiling_on_sc=False, needs_layout_passes=False),
        out_type=jax.ShapeDtypeStruct((SEQ, D // 8, IB, 8 * CHUNK), jnp.float32),
        scratch_types=[
            pltpu.VMEM((nchunk, CHUNK), jnp.int32),
            pltpu.VMEM((NBUF, CHUNK, D), jnp.float32),
            pltpu.VMEM((NBUF, D // 8 * 8 * CHUNK), jnp.float32),
        ] + [pltpu.SemaphoreType.DMA] * (2 * NBUF),
    )
    def sc_kernel(x_hbm, tab_hbm, out_hbm, idx_v, a_v, c_v, *sems):
        sga = sems[0:NBUF]
        so = sems[NBUF:2 * NBUF]

        wid = lax.axis_index("s") * 2 + lax.axis_index("c")
        pltpu.sync_copy(x_hbm.at[wid], idx_v)
        kbase = wid * nchunk

        iot = lax.iota(jnp.int32, LANES)
        rowvs = [iot + (g * LANES) for g in range(CHUNK // LANES)]

        def issue_gather(n, s):
            pltpu.async_copy(tab_hbm.at[idx_v.at[n]], a_v.at[s], sga[s])

        def issue_out(n, s):
            k = kbase + n
            j = k // IB
            ib = k % IB
            for q in range(D // 8):
                pltpu.async_copy(
                    c_v.at[s, pl.ds(q * 8 * CHUNK, 8 * CHUNK)],
                    out_hbm.at[j, q, ib], so[s])

        def wait_out(s):
            for q in range(D // 8):
                pltpu.make_async_copy(
                    c_v.at[s, pl.ds(q * 8 * CHUNK, 8 * CHUNK)],
                    out_hbm.at[0, q, 0], so[s]).wait()

        for s in range(NBUF):
            issue_gather(s, s)

        def round_body(g, carry):
            for s in range(NBUF):
                n = g * NBUF + s
                pltpu.make_async_copy(
                    tab_hbm.at[idx_v.at[n]], a_v.at[s], sga[s]).wait()

                def col_body(c0, c2):
                    colv = (c0 + iot) & (D - 1)
                    cpart = ((colv >> 3) << 10) + ((colv & 7) << 7)
                    for gr in range(CHUNK // LANES):
                        av = plsc.load_gather(a_v.at[s], [rowvs[gr], colv])
                        plsc.store_scatter(
                            c_v.at[s], [cpart + rowvs[gr]], av)
                    return c2

                lax.fori_loop(0, D, col_body, 0)

                @pl.when(g < nrounds - 1)
                def _():
                    issue_gather(n + NBUF, s)

                @pl.when(g > 0)
                def _():
                    wait_out(s)

                issue_out(n, s)
            return carry

        lax.fori_loop(0, nrounds, round_body, 0)
        for s in range(NBUF):
            wait_out(s)

    return sc_kernel


def kernel(x, table_1, table_2):
    rows, seq = x.shape
    flat = rows * seq
    per_w = flat // NW
    nchunk = per_w // CHUNK
    sub = TC_BK // 4
    w = x % TC_BK
    xp = (x - w) + (w % sub) * 4 + w // sub
    xf = xp.T.reshape(NW, nchunk, CHUNK)
    summed = _tc_sum_transpose(table_1.T, table_2.T)
    out = _make_sc_kernel(nchunk)(
        xf, summed.reshape(summed.shape[0] * 4, D))
    out = out.reshape(SEQ, D // 8, IB, 8, CHUNK)
    return out.transpose(2, 4, 0, 1, 3).reshape(rows, seq, D)
